# baseline (speedup 1.0000x reference)
import functools

import jax
import jax.numpy as jnp
from jax import lax
from jax.experimental import pallas as pl
from jax.experimental.pallas import tpu as pltpu
from jax.experimental.pallas import tpu_sc as plsc

NC = 2
NS = 16
NW = NC * NS


def _make_gather(V, D, B, C, NB, NSB, P2):
    assert B % NW == 0
    b_per_w = B // NW
    assert b_per_w % C == 0
    chunks = b_per_w // C
    P1 = NB - P2
    assert chunks % NB == 0 and chunks >= 2 * NB and 0 < P2 < NSB
    assert NB % NSB == 0
    mesh = plsc.VectorSubcoreMesh(core_axis_name="c", subcore_axis_name="s")

    scratch = [pltpu.VMEM((b_per_w,), jnp.int32)]
    scratch += [pltpu.VMEM((C, D), jnp.float32) for _ in range(NB)]
    scratch += [pltpu.VMEM_SHARED((NS, NSB, C, D), jnp.float32)]
    scratch += [pltpu.SemaphoreType.DMA for _ in range(2 * NB + NSB)]

    @functools.partial(
        pl.kernel,
        mesh=mesh,
        out_type=jax.ShapeDtypeStruct((B, D), jnp.float32),
        scratch_types=scratch,
    )
    def gather_kernel(table_hbm, idx_hbm, out_hbm, idx_v, *rest):
        bufs = rest[:NB]
        shared = rest[NB]
        in_sems = rest[NB + 1:2 * NB + 1]
        x_sems = rest[2 * NB + 1:3 * NB + 1]
        out_sems = rest[3 * NB + 1:]
        sid = lax.axis_index("s")
        wid = sid * NC + lax.axis_index("c")
        base = wid * b_per_w
        pltpu.sync_copy(idx_hbm.at[pl.ds(base, b_per_w)], idx_v)

        def slot(b):
            return shared.at[sid, b % NSB]

        def out_slice(c):
            return out_hbm.at[pl.ds(base + c * C, C)]

        def gather_into(c, b):
            pltpu.async_copy(
                table_hbm.at[idx_v.at[pl.ds(c * C, C)]], bufs[b], in_sems[b])

        for j in range(P1):
            gather_into(j, j)

        def body(i, carry):
            g = i * NB
            for b in range(NB):
                c = g + b
                pltpu.make_async_copy(
                    table_hbm.at[idx_v.at[pl.ds(c * C, C)]],
                    bufs[b], in_sems[b]).wait()

                @pl.when(c >= NSB)
                def _():
                    pltpu.make_async_copy(
                        slot(b), out_slice(c - NSB),
                        out_sems[b % NSB]).wait()

                pltpu.async_copy(bufs[b], slot(b), x_sems[b])

                cp = c - P2
                bp = (b + NB - P2) % NB

                @pl.when(cp >= 0)
                def _():
                    pltpu.make_async_copy(
                        bufs[bp], slot(bp), x_sems[bp]).wait()
                    pltpu.async_copy(
                        slot(bp), out_slice(cp), out_sems[bp % NSB])

                f = c + P1

                @pl.when(f < chunks)
                def _():
                    gather_into(f, bp)
            return carry

        lax.fori_loop(0, chunks // NB, body, 0)

        for c in range(chunks - P2, chunks):
            b = c % NB
            pltpu.make_async_copy(bufs[b], slot(b), x_sems[b]).wait()
            pltpu.async_copy(slot(b), out_slice(c), out_sems[b % NSB])
        for c in range(chunks - NSB, chunks):
            b = c % NB
            pltpu.make_async_copy(
                slot(b), out_slice(c), out_sems[b % NSB]).wait()

    return gather_kernel


def kernel(position_ids, pe_weight):
    V, D = pe_weight.shape
    orig_shape = position_ids.shape
    B = position_ids.size
    C, NB, NSB, P2 = 8, 8, 4, 2
    idx = position_ids.astype(jnp.int32).reshape(B)
    out = _make_gather(V, D, B, C, NB, NSB, P2)(pe_weight, idx)
    return out.reshape(orig_shape + (D,))
